# baseline (speedup 1.0000x reference)
import functools

import jax
import jax.numpy as jnp
from jax import lax
from jax.experimental import pallas as pl
from jax.experimental.pallas import tpu as pltpu
from jax.experimental.pallas import tpu_sc as plsc

N = 50000
D = 256
S = 512
NC, NS = 2, 16
NW = NC * NS
SEG_T = S // NW
C = 224
NBUF = 2
NV = D // 16
STRIDE = 64
NSMP = (N + STRIDE - 1) // STRIDE
WIN = 72
SMP_STEPS = 10
WIN_STEPS = 7


def kernel(x, batch):
    batch_i32 = batch.astype(jnp.int32)
    sample = batch_i32[::STRIDE]
    mesh = plsc.VectorSubcoreMesh(core_axis_name="c", subcore_axis_name="s")

    @functools.partial(
        pl.kernel,
        mesh=mesh,
        compiler_params=pltpu.CompilerParams(needs_layout_passes=False),
        out_type=jax.ShapeDtypeStruct((S, D), jnp.float32),
        scratch_types=(
            [pltpu.VMEM((NSMP,), jnp.int32),
             pltpu.VMEM(((SEG_T + 1) * WIN,), jnp.int32)]
            + [pltpu.VMEM((C, D), jnp.float32)] * NBUF
            + [pltpu.VMEM((SEG_T, D), jnp.float32),
               pltpu.SMEM((SEG_T + 1,), jnp.int32)]
            + [pltpu.SemaphoreType.DMA] * (NBUF + 1)
        ),
    )
    def k(x_hbm, b_hbm, smp_hbm, out_hbm, smp_v, win_v, *rest):
        bufs = rest[:NBUF]
        acc_v, bnd_s = rest[NBUF], rest[NBUF + 1]
        sems = rest[NBUF + 2:NBUF + 2 + NBUF]
        wsem = rest[NBUF + 2 + NBUF]
        c = lax.axis_index("c")
        s = lax.axis_index("s")
        w = c * NS + s
        seg0 = w * SEG_T

        pltpu.sync_copy(smp_hbm, smp_v)

        lane = lax.iota(jnp.int32, 16)

        def lower_bound_smp(tgt):
            def step(_, lh):
                lo, hi = lh
                active = lo < hi
                mid = jnp.minimum((lo + hi) // 2, NSMP - 1)
                vals = plsc.load_gather(smp_v, [mid])
                pred = vals < tgt
                lo = jnp.where(active & pred, mid + 1, lo)
                hi = jnp.where(active & (~pred), mid, hi)
                return lo, hi
            lo, _ = lax.fori_loop(
                0, SMP_STEPS, step,
                (jnp.zeros((16,), jnp.int32),
                 jnp.full((16,), NSMP, jnp.int32)))
            return lo

        coarse_lo = lower_bound_smp(seg0 + lane)
        coarse_up = lower_bound_smp(seg0 + 1 + lane)
        base_lo = jnp.clip(STRIDE * (coarse_lo - 1), 0, N - WIN)
        base_up = jnp.clip(STRIDE * (coarse_up - 1), 0, N - WIN)

        def extract(vec, idx):
            return jnp.sum(jnp.where(lane == idx, vec, 0))

        for j in range(SEG_T):
            pltpu.async_copy(
                b_hbm.at[pl.ds(pl.multiple_of(extract(base_lo, j), 8), WIN)],
                win_v.at[pl.ds(j * WIN, WIN)], wsem)
        pltpu.async_copy(
            b_hbm.at[pl.ds(pl.multiple_of(extract(base_up, SEG_T - 1), 8),
                           WIN)],
            win_v.at[pl.ds(SEG_T * WIN, WIN)], wsem)
        for j in range(SEG_T + 1):
            pltpu.make_async_copy(b_hbm.at[pl.ds(0, WIN)],
                                  win_v.at[pl.ds(0, WIN)], wsem).wait()

        def lower_bound_win(rows, tgt):
            def step(_, lh):
                lo, hi = lh
                active = lo < hi
                mid = (lo + hi) // 2
                vals = plsc.load_gather(win_v, [rows * WIN + mid])
                pred = vals < tgt
                lo = jnp.where(active & pred, mid + 1, lo)
                hi = jnp.where(active & (~pred), mid, hi)
                return lo, hi
            lo, _ = lax.fori_loop(
                0, WIN_STEPS, step,
                (jnp.zeros((16,), jnp.int32),
                 jnp.full((16,), WIN, jnp.int32)))
            return lo

        lob = base_lo + lower_bound_win(lane, seg0 + lane)
        upb = base_up + lower_bound_win(lane + 1, seg0 + 1 + lane)
        recips = jnp.ones((16,), jnp.float32) / jnp.maximum(
            (upb - lob).astype(jnp.float32), 1.0)

        lo_w = extract(lob, 0)
        hi_w = extract(upb, SEG_T - 1)
        bnd_s[0] = lo_w
        for kk in range(SEG_T):
            bnd_s[kk + 1] = extract(upb, kk)

        a0 = pl.multiple_of(jnp.minimum((lo_w // 8) * 8, N - C), 8)
        nch = (hi_w - a0 + C - 1) // C

        def chunk_base(i):
            return pl.multiple_of(
                jnp.minimum(a0 + i * C, N - C), 8)

        def issue(i, parity):
            @pl.when(i < nch)
            def _():
                pltpu.async_copy(x_hbm.at[pl.ds(chunk_base(i), C)],
                                 bufs[parity], sems[parity])

        def drain(parity):
            pltpu.make_async_copy(x_hbm.at[pl.ds(0, C)], bufs[parity],
                                  sems[parity]).wait()

        for b in range(NBUF):
            issue(b, b)

        zero = jnp.zeros((16,), jnp.float32)

        def process(i, parity, st):
            buf = bufs[parity]
            bc = chunk_base(i)
            ce = jnp.minimum(hi_w, bc + C)

            def piece_cond(pst):
                return pst[0] < ce

            def piece_body(pst):
                p, kseg = pst[0], pst[1]
                acc = pst[2:]
                b_next = bnd_s[kseg + 1]
                e = jnp.minimum(b_next, ce)

                p0 = p - bc
                n = e - p

                def row2(i, a):
                    r = p0 + 2 * i
                    return tuple(a[j] + buf[r, pl.ds(j * 16, 16)]
                                 + buf[r + 1, pl.ds(j * 16, 16)]
                                 for j in range(NV))
                acc = lax.fori_loop(0, n // 2, row2, acc)
                lr = jnp.maximum(e - 1 - bc, 0)
                sv = jnp.full((16,), jnp.where((n & 1) == 1, 1.0, 0.0))
                acc = tuple(acc[j] + buf[lr, pl.ds(j * 16, 16)] * sv
                            for j in range(NV))
                flushed = e == b_next

                @pl.when(flushed)
                def _():
                    recip = jnp.full((16,), extract(recips, kseg))
                    for j in range(NV):
                        acc_v[kseg, pl.ds(j * 16, 16)] = acc[j] * recip

                acc = tuple(jnp.where(flushed, zero, a) for a in acc)
                return (e, kseg + flushed.astype(jnp.int32)) + acc

            return lax.while_loop(piece_cond, piece_body, st)

        def loop_cond(st):
            return st[0] < nch

        def loop_body(st):
            i = st[0]
            pst = st[1:]
            for b in range(NBUF):
                if b == 0:
                    drain(0)
                else:
                    @pl.when(i + b < nch)
                    def _(b=b):
                        drain(b)
                pst = process(i + b, b, pst)
                issue(i + b + NBUF, b)
            return (i + NBUF,) + pst

        st = lax.while_loop(loop_cond, loop_body,
                            (0, lo_w, 0) + (zero,) * NV)
        kseg_end = st[2]

        def tail_cond(kk):
            return kk < SEG_T

        def tail_body(kk):
            for j in range(NV):
                acc_v[kk, pl.ds(j * 16, 16)] = zero
            return kk + 1

        lax.while_loop(tail_cond, tail_body, kseg_end)

        pltpu.sync_copy(acc_v, out_hbm.at[pl.ds(pl.multiple_of(seg0, 8),
                                                SEG_T)])

    return k(x, batch_i32, sample)


# speedup vs baseline: 1.0149x; 1.0149x over previous
import functools

import jax
import jax.numpy as jnp
from jax import lax
from jax.experimental import pallas as pl
from jax.experimental.pallas import tpu as pltpu
from jax.experimental.pallas import tpu_sc as plsc

N = 50000
D = 256
S = 512
NC, NS = 2, 16
NW = NC * NS
SEG_T = S // NW
C = 96
NBUF = 5
NV = D // 16
STRIDE = 64
NSMP = (N + STRIDE - 1) // STRIDE
WIN = 72
SMP_STEPS = 10
WIN_STEPS = 7


def kernel(x, batch):
    batch_i32 = batch.astype(jnp.int32)
    sample = batch_i32[::STRIDE]
    mesh = plsc.VectorSubcoreMesh(core_axis_name="c", subcore_axis_name="s")

    @functools.partial(
        pl.kernel,
        mesh=mesh,
        compiler_params=pltpu.CompilerParams(needs_layout_passes=False),
        out_type=jax.ShapeDtypeStruct((S, D), jnp.float32),
        scratch_types=(
            [pltpu.VMEM((NSMP,), jnp.int32),
             pltpu.VMEM(((SEG_T + 1) * WIN,), jnp.int32)]
            + [pltpu.VMEM((C, D), jnp.float32)] * NBUF
            + [pltpu.VMEM((SEG_T, D), jnp.float32),
               pltpu.SMEM((SEG_T + 1,), jnp.int32)]
            + [pltpu.SemaphoreType.DMA] * (NBUF + 1)
        ),
    )
    def k(x_hbm, b_hbm, smp_hbm, out_hbm, smp_v, win_v, *rest):
        bufs = rest[:NBUF]
        acc_v, bnd_s = rest[NBUF], rest[NBUF + 1]
        sems = rest[NBUF + 2:NBUF + 2 + NBUF]
        wsem = rest[NBUF + 2 + NBUF]
        c = lax.axis_index("c")
        s = lax.axis_index("s")
        w = c * NS + s
        seg0 = w * SEG_T

        pltpu.sync_copy(smp_hbm, smp_v)

        lane = lax.iota(jnp.int32, 16)

        def lower_bound_smp(tgt):
            def step(_, lh):
                lo, hi = lh
                active = lo < hi
                mid = jnp.minimum((lo + hi) // 2, NSMP - 1)
                vals = plsc.load_gather(smp_v, [mid])
                pred = vals < tgt
                lo = jnp.where(active & pred, mid + 1, lo)
                hi = jnp.where(active & (~pred), mid, hi)
                return lo, hi
            lo, _ = lax.fori_loop(
                0, SMP_STEPS, step,
                (jnp.zeros((16,), jnp.int32),
                 jnp.full((16,), NSMP, jnp.int32)))
            return lo

        coarse_lo = lower_bound_smp(seg0 + lane)
        coarse_up = lower_bound_smp(seg0 + 1 + lane)
        base_lo = jnp.clip(STRIDE * (coarse_lo - 1), 0, N - WIN)
        base_up = jnp.clip(STRIDE * (coarse_up - 1), 0, N - WIN)

        def extract(vec, idx):
            return jnp.sum(jnp.where(lane == idx, vec, 0))

        for j in range(SEG_T):
            pltpu.async_copy(
                b_hbm.at[pl.ds(pl.multiple_of(extract(base_lo, j), 8), WIN)],
                win_v.at[pl.ds(j * WIN, WIN)], wsem)
        pltpu.async_copy(
            b_hbm.at[pl.ds(pl.multiple_of(extract(base_up, SEG_T - 1), 8),
                           WIN)],
            win_v.at[pl.ds(SEG_T * WIN, WIN)], wsem)
        for j in range(SEG_T + 1):
            pltpu.make_async_copy(b_hbm.at[pl.ds(0, WIN)],
                                  win_v.at[pl.ds(0, WIN)], wsem).wait()

        def lower_bound_win(rows, tgt):
            def step(_, lh):
                lo, hi = lh
                active = lo < hi
                mid = (lo + hi) // 2
                vals = plsc.load_gather(win_v, [rows * WIN + mid])
                pred = vals < tgt
                lo = jnp.where(active & pred, mid + 1, lo)
                hi = jnp.where(active & (~pred), mid, hi)
                return lo, hi
            lo, _ = lax.fori_loop(
                0, WIN_STEPS, step,
                (jnp.zeros((16,), jnp.int32),
                 jnp.full((16,), WIN, jnp.int32)))
            return lo

        lob = base_lo + lower_bound_win(lane, seg0 + lane)
        upb = base_up + lower_bound_win(lane + 1, seg0 + 1 + lane)
        recips = jnp.ones((16,), jnp.float32) / jnp.maximum(
            (upb - lob).astype(jnp.float32), 1.0)

        lo_w = extract(lob, 0)
        hi_w = extract(upb, SEG_T - 1)
        bnd_s[0] = lo_w
        for kk in range(SEG_T):
            bnd_s[kk + 1] = extract(upb, kk)

        a0 = pl.multiple_of(jnp.minimum((lo_w // 8) * 8, N - C), 8)
        nch = (hi_w - a0 + C - 1) // C

        def chunk_base(i):
            return pl.multiple_of(
                jnp.minimum(a0 + i * C, N - C), 8)

        def issue(i, parity):
            @pl.when(i < nch)
            def _():
                pltpu.async_copy(x_hbm.at[pl.ds(chunk_base(i), C)],
                                 bufs[parity], sems[parity])

        def drain(parity):
            pltpu.make_async_copy(x_hbm.at[pl.ds(0, C)], bufs[parity],
                                  sems[parity]).wait()

        for b in range(NBUF):
            issue(b, b)

        zero = jnp.zeros((16,), jnp.float32)

        def process(i, parity, st):
            buf = bufs[parity]
            bc = chunk_base(i)
            ce = jnp.minimum(hi_w, bc + C)

            def piece_cond(pst):
                return pst[0] < ce

            def piece_body(pst):
                p, kseg = pst[0], pst[1]
                acc = pst[2:]
                b_next = bnd_s[kseg + 1]
                e = jnp.minimum(b_next, ce)

                p0 = p - bc
                n = e - p

                def row2(i, a):
                    r = p0 + 2 * i
                    return tuple(a[j] + buf[r, pl.ds(j * 16, 16)]
                                 + buf[r + 1, pl.ds(j * 16, 16)]
                                 for j in range(NV))
                acc = lax.fori_loop(0, n // 2, row2, acc)
                lr = jnp.maximum(e - 1 - bc, 0)
                sv = jnp.full((16,), jnp.where((n & 1) == 1, 1.0, 0.0))
                acc = tuple(acc[j] + buf[lr, pl.ds(j * 16, 16)] * sv
                            for j in range(NV))
                flushed = e == b_next

                @pl.when(flushed)
                def _():
                    recip = jnp.full((16,), extract(recips, kseg))
                    for j in range(NV):
                        acc_v[kseg, pl.ds(j * 16, 16)] = acc[j] * recip

                acc = tuple(jnp.where(flushed, zero, a) for a in acc)
                return (e, kseg + flushed.astype(jnp.int32)) + acc

            return lax.while_loop(piece_cond, piece_body, st)

        def loop_cond(st):
            return st[0] < nch

        def loop_body(st):
            i = st[0]
            pst = st[1:]
            for b in range(NBUF):
                if b == 0:
                    drain(0)
                else:
                    @pl.when(i + b < nch)
                    def _(b=b):
                        drain(b)
                pst = process(i + b, b, pst)
                issue(i + b + NBUF, b)
            return (i + NBUF,) + pst

        st = lax.while_loop(loop_cond, loop_body,
                            (0, lo_w, 0) + (zero,) * NV)
        kseg_end = st[2]

        def tail_cond(kk):
            return kk < SEG_T

        def tail_body(kk):
            for j in range(NV):
                acc_v[kk, pl.ds(j * 16, 16)] = zero
            return kk + 1

        lax.while_loop(tail_cond, tail_body, kseg_end)

        pltpu.sync_copy(acc_v, out_hbm.at[pl.ds(pl.multiple_of(seg0, 8),
                                                SEG_T)])

    return k(x, batch_i32, sample)


# speedup vs baseline: 1.0242x; 1.0091x over previous
import functools

import jax
import jax.numpy as jnp
from jax import lax
from jax.experimental import pallas as pl
from jax.experimental.pallas import tpu as pltpu
from jax.experimental.pallas import tpu_sc as plsc

N = 50000
D = 256
S = 512
NC, NS = 2, 16
NW = NC * NS
SEG_T = S // NW
C = 120
NBUF = 4
NV = D // 16
STRIDE = 64
NSMP = (N + STRIDE - 1) // STRIDE
WIN = 72
SMP_STEPS = 10
WIN_STEPS = 7


def kernel(x, batch):
    batch_i32 = batch.astype(jnp.int32)
    sample = batch_i32[::STRIDE]
    mesh = plsc.VectorSubcoreMesh(core_axis_name="c", subcore_axis_name="s")

    @functools.partial(
        pl.kernel,
        mesh=mesh,
        compiler_params=pltpu.CompilerParams(needs_layout_passes=False),
        out_type=jax.ShapeDtypeStruct((S, D), jnp.float32),
        scratch_types=(
            [pltpu.VMEM((NSMP,), jnp.int32),
             pltpu.VMEM(((SEG_T + 1) * WIN,), jnp.int32)]
            + [pltpu.VMEM((C, D), jnp.float32)] * NBUF
            + [pltpu.VMEM((SEG_T, D), jnp.float32),
               pltpu.SMEM((SEG_T + 1,), jnp.int32)]
            + [pltpu.SemaphoreType.DMA] * (NBUF + 1)
        ),
    )
    def k(x_hbm, b_hbm, smp_hbm, out_hbm, smp_v, win_v, *rest):
        bufs = rest[:NBUF]
        acc_v, bnd_s = rest[NBUF], rest[NBUF + 1]
        sems = rest[NBUF + 2:NBUF + 2 + NBUF]
        wsem = rest[NBUF + 2 + NBUF]
        c = lax.axis_index("c")
        s = lax.axis_index("s")
        w = c * NS + s
        seg0 = w * SEG_T

        pltpu.sync_copy(smp_hbm, smp_v)

        lane = lax.iota(jnp.int32, 16)

        def lower_bound_smp(tgt):
            def step(_, lh):
                lo, hi = lh
                active = lo < hi
                mid = jnp.minimum((lo + hi) // 2, NSMP - 1)
                vals = plsc.load_gather(smp_v, [mid])
                pred = vals < tgt
                lo = jnp.where(active & pred, mid + 1, lo)
                hi = jnp.where(active & (~pred), mid, hi)
                return lo, hi
            lo, _ = lax.fori_loop(
                0, SMP_STEPS, step,
                (jnp.zeros((16,), jnp.int32),
                 jnp.full((16,), NSMP, jnp.int32)))
            return lo

        coarse_lo = lower_bound_smp(seg0 + lane)
        coarse_up = lower_bound_smp(seg0 + 1 + lane)
        base_lo = jnp.clip(STRIDE * (coarse_lo - 1), 0, N - WIN)
        base_up = jnp.clip(STRIDE * (coarse_up - 1), 0, N - WIN)

        def extract(vec, idx):
            return jnp.sum(jnp.where(lane == idx, vec, 0))

        for j in range(SEG_T):
            pltpu.async_copy(
                b_hbm.at[pl.ds(pl.multiple_of(extract(base_lo, j), 8), WIN)],
                win_v.at[pl.ds(j * WIN, WIN)], wsem)
        pltpu.async_copy(
            b_hbm.at[pl.ds(pl.multiple_of(extract(base_up, SEG_T - 1), 8),
                           WIN)],
            win_v.at[pl.ds(SEG_T * WIN, WIN)], wsem)
        for j in range(SEG_T + 1):
            pltpu.make_async_copy(b_hbm.at[pl.ds(0, WIN)],
                                  win_v.at[pl.ds(0, WIN)], wsem).wait()

        def lower_bound_win(rows, tgt):
            def step(_, lh):
                lo, hi = lh
                active = lo < hi
                mid = (lo + hi) // 2
                vals = plsc.load_gather(win_v, [rows * WIN + mid])
                pred = vals < tgt
                lo = jnp.where(active & pred, mid + 1, lo)
                hi = jnp.where(active & (~pred), mid, hi)
                return lo, hi
            lo, _ = lax.fori_loop(
                0, WIN_STEPS, step,
                (jnp.zeros((16,), jnp.int32),
                 jnp.full((16,), WIN, jnp.int32)))
            return lo

        lob = base_lo + lower_bound_win(lane, seg0 + lane)
        upb = base_up + lower_bound_win(lane + 1, seg0 + 1 + lane)
        recips = jnp.ones((16,), jnp.float32) / jnp.maximum(
            (upb - lob).astype(jnp.float32), 1.0)

        lo_w = extract(lob, 0)
        hi_w = extract(upb, SEG_T - 1)
        bnd_s[0] = lo_w
        for kk in range(SEG_T):
            bnd_s[kk + 1] = extract(upb, kk)

        a0 = pl.multiple_of(jnp.minimum((lo_w // 8) * 8, N - C), 8)
        nch = (hi_w - a0 + C - 1) // C

        def chunk_base(i):
            return pl.multiple_of(
                jnp.minimum(a0 + i * C, N - C), 8)

        def issue(i, parity):
            @pl.when(i < nch)
            def _():
                pltpu.async_copy(x_hbm.at[pl.ds(chunk_base(i), C)],
                                 bufs[parity], sems[parity])

        def drain(parity):
            pltpu.make_async_copy(x_hbm.at[pl.ds(0, C)], bufs[parity],
                                  sems[parity]).wait()

        for b in range(NBUF):
            issue(b, b)

        zero = jnp.zeros((16,), jnp.float32)

        def process(i, parity, st):
            buf = bufs[parity]
            bc = chunk_base(i)
            ce = jnp.minimum(hi_w, bc + C)

            def piece_cond(pst):
                return pst[0] < ce

            def piece_body(pst):
                p, kseg = pst[0], pst[1]
                acc = pst[2:]
                b_next = bnd_s[kseg + 1]
                e = jnp.minimum(b_next, ce)

                p0 = p - bc
                n = e - p

                def row2(i, a):
                    r = p0 + 2 * i
                    return tuple(a[j] + buf[r, pl.ds(j * 16, 16)]
                                 + buf[r + 1, pl.ds(j * 16, 16)]
                                 for j in range(NV))
                acc = lax.fori_loop(0, n // 2, row2, acc)
                lr = jnp.maximum(e - 1 - bc, 0)
                sv = jnp.full((16,), jnp.where((n & 1) == 1, 1.0, 0.0))
                acc = tuple(acc[j] + buf[lr, pl.ds(j * 16, 16)] * sv
                            for j in range(NV))
                flushed = e == b_next

                @pl.when(flushed)
                def _():
                    recip = jnp.full((16,), extract(recips, kseg))
                    for j in range(NV):
                        acc_v[kseg, pl.ds(j * 16, 16)] = acc[j] * recip

                acc = tuple(jnp.where(flushed, zero, a) for a in acc)
                return (e, kseg + flushed.astype(jnp.int32)) + acc

            return lax.while_loop(piece_cond, piece_body, st)

        def loop_cond(st):
            return st[0] < nch

        def loop_body(st):
            i = st[0]
            pst = st[1:]
            for b in range(NBUF):
                if b == 0:
                    drain(0)
                else:
                    @pl.when(i + b < nch)
                    def _(b=b):
                        drain(b)
                pst = process(i + b, b, pst)
                issue(i + b + NBUF, b)
            return (i + NBUF,) + pst

        st = lax.while_loop(loop_cond, loop_body,
                            (0, lo_w, 0) + (zero,) * NV)
        kseg_end = st[2]

        def tail_cond(kk):
            return kk < SEG_T

        def tail_body(kk):
            for j in range(NV):
                acc_v[kk, pl.ds(j * 16, 16)] = zero
            return kk + 1

        lax.while_loop(tail_cond, tail_body, kseg_end)

        pltpu.sync_copy(acc_v, out_hbm.at[pl.ds(pl.multiple_of(seg0, 8),
                                                SEG_T)])

    return k(x, batch_i32, sample)


# speedup vs baseline: 1.0293x; 1.0051x over previous
import functools

import jax
import jax.numpy as jnp
from jax import lax
from jax.experimental import pallas as pl
from jax.experimental.pallas import tpu as pltpu
from jax.experimental.pallas import tpu_sc as plsc

N = 50000
D = 256
S = 512
NC, NS = 2, 16
NW = NC * NS
SEG_T = S // NW
C = 112
NBUF = 4
NV = D // 16
STRIDE = 64
NSMP = (N + STRIDE - 1) // STRIDE
WIN = 72
SMP_STEPS = 10
WIN_STEPS = 7


def kernel(x, batch):
    batch_i32 = batch.astype(jnp.int32)
    sample = batch_i32[::STRIDE]
    mesh = plsc.VectorSubcoreMesh(core_axis_name="c", subcore_axis_name="s")

    @functools.partial(
        pl.kernel,
        mesh=mesh,
        compiler_params=pltpu.CompilerParams(needs_layout_passes=False),
        out_type=jax.ShapeDtypeStruct((S, D), jnp.float32),
        scratch_types=(
            [pltpu.VMEM((NSMP,), jnp.int32),
             pltpu.VMEM(((SEG_T + 1) * WIN,), jnp.int32)]
            + [pltpu.VMEM((C, D), jnp.float32)] * NBUF
            + [pltpu.VMEM((SEG_T, D), jnp.float32),
               pltpu.SMEM((SEG_T + 1,), jnp.int32)]
            + [pltpu.SemaphoreType.DMA] * (NBUF + 1)
        ),
    )
    def k(x_hbm, b_hbm, smp_hbm, out_hbm, smp_v, win_v, *rest):
        bufs = rest[:NBUF]
        acc_v, bnd_s = rest[NBUF], rest[NBUF + 1]
        sems = rest[NBUF + 2:NBUF + 2 + NBUF]
        wsem = rest[NBUF + 2 + NBUF]
        c = lax.axis_index("c")
        s = lax.axis_index("s")
        w = c * NS + s
        seg0 = w * SEG_T

        pltpu.sync_copy(smp_hbm, smp_v)

        lane = lax.iota(jnp.int32, 16)

        def lower_bound_smp(tgt):
            def step(_, lh):
                lo, hi = lh
                active = lo < hi
                mid = jnp.minimum((lo + hi) // 2, NSMP - 1)
                vals = plsc.load_gather(smp_v, [mid])
                pred = vals < tgt
                lo = jnp.where(active & pred, mid + 1, lo)
                hi = jnp.where(active & (~pred), mid, hi)
                return lo, hi
            lo, _ = lax.fori_loop(
                0, SMP_STEPS, step,
                (jnp.zeros((16,), jnp.int32),
                 jnp.full((16,), NSMP, jnp.int32)))
            return lo

        coarse_lo = lower_bound_smp(seg0 + lane)
        coarse_up = lower_bound_smp(seg0 + 1 + lane)
        base_lo = jnp.clip(STRIDE * (coarse_lo - 1), 0, N - WIN)
        base_up = jnp.clip(STRIDE * (coarse_up - 1), 0, N - WIN)

        def extract(vec, idx):
            return jnp.sum(jnp.where(lane == idx, vec, 0))

        for j in range(SEG_T):
            pltpu.async_copy(
                b_hbm.at[pl.ds(pl.multiple_of(extract(base_lo, j), 8), WIN)],
                win_v.at[pl.ds(j * WIN, WIN)], wsem)
        pltpu.async_copy(
            b_hbm.at[pl.ds(pl.multiple_of(extract(base_up, SEG_T - 1), 8),
                           WIN)],
            win_v.at[pl.ds(SEG_T * WIN, WIN)], wsem)
        for j in range(SEG_T + 1):
            pltpu.make_async_copy(b_hbm.at[pl.ds(0, WIN)],
                                  win_v.at[pl.ds(0, WIN)], wsem).wait()

        def lower_bound_win(rows, tgt):
            def step(_, lh):
                lo, hi = lh
                active = lo < hi
                mid = (lo + hi) // 2
                vals = plsc.load_gather(win_v, [rows * WIN + mid])
                pred = vals < tgt
                lo = jnp.where(active & pred, mid + 1, lo)
                hi = jnp.where(active & (~pred), mid, hi)
                return lo, hi
            lo, _ = lax.fori_loop(
                0, WIN_STEPS, step,
                (jnp.zeros((16,), jnp.int32),
                 jnp.full((16,), WIN, jnp.int32)))
            return lo

        lob = base_lo + lower_bound_win(lane, seg0 + lane)
        upb = base_up + lower_bound_win(lane + 1, seg0 + 1 + lane)
        recips = jnp.ones((16,), jnp.float32) / jnp.maximum(
            (upb - lob).astype(jnp.float32), 1.0)

        lo_w = extract(lob, 0)
        hi_w = extract(upb, SEG_T - 1)
        bnd_s[0] = lo_w
        for kk in range(SEG_T):
            bnd_s[kk + 1] = extract(upb, kk)

        a0 = pl.multiple_of(jnp.minimum((lo_w // 8) * 8, N - C), 8)
        nch = (hi_w - a0 + C - 1) // C

        def chunk_base(i):
            return pl.multiple_of(
                jnp.minimum(a0 + i * C, N - C), 8)

        def issue(i, parity):
            @pl.when(i < nch)
            def _():
                pltpu.async_copy(x_hbm.at[pl.ds(chunk_base(i), C)],
                                 bufs[parity], sems[parity])

        def drain(parity):
            pltpu.make_async_copy(x_hbm.at[pl.ds(0, C)], bufs[parity],
                                  sems[parity]).wait()

        for b in range(NBUF):
            issue(b, b)

        zero = jnp.zeros((16,), jnp.float32)

        def process(i, parity, st):
            buf = bufs[parity]
            bc = chunk_base(i)
            ce = jnp.minimum(hi_w, bc + C)

            def piece_cond(pst):
                return pst[0] < ce

            def piece_body(pst):
                p, kseg = pst[0], pst[1]
                acc = pst[2:]
                b_next = bnd_s[kseg + 1]
                e = jnp.minimum(b_next, ce)

                p0 = p - bc
                n = e - p

                def row2(i, a):
                    r = p0 + 2 * i
                    return tuple(a[j] + buf[r, pl.ds(j * 16, 16)]
                                 + buf[r + 1, pl.ds(j * 16, 16)]
                                 for j in range(NV))
                acc = lax.fori_loop(0, n // 2, row2, acc)
                lr = jnp.maximum(e - 1 - bc, 0)
                sv = jnp.full((16,), jnp.where((n & 1) == 1, 1.0, 0.0))
                acc = tuple(acc[j] + buf[lr, pl.ds(j * 16, 16)] * sv
                            for j in range(NV))
                flushed = e == b_next

                @pl.when(flushed)
                def _():
                    recip = jnp.full((16,), extract(recips, kseg))
                    for j in range(NV):
                        acc_v[kseg, pl.ds(j * 16, 16)] = acc[j] * recip

                acc = tuple(jnp.where(flushed, zero, a) for a in acc)
                return (e, kseg + flushed.astype(jnp.int32)) + acc

            return lax.while_loop(piece_cond, piece_body, st)

        def loop_cond(st):
            return st[0] < nch

        def loop_body(st):
            i = st[0]
            pst = st[1:]
            for b in range(NBUF):
                if b == 0:
                    drain(0)
                else:
                    @pl.when(i + b < nch)
                    def _(b=b):
                        drain(b)
                pst = process(i + b, b, pst)
                issue(i + b + NBUF, b)
            return (i + NBUF,) + pst

        st = lax.while_loop(loop_cond, loop_body,
                            (0, lo_w, 0) + (zero,) * NV)
        kseg_end = st[2]

        def tail_cond(kk):
            return kk < SEG_T

        def tail_body(kk):
            for j in range(NV):
                acc_v[kk, pl.ds(j * 16, 16)] = zero
            return kk + 1

        lax.while_loop(tail_cond, tail_body, kseg_end)

        pltpu.sync_copy(acc_v, out_hbm.at[pl.ds(pl.multiple_of(seg0, 8),
                                                SEG_T)])

    return k(x, batch_i32, sample)
